# baseline (speedup 1.0000x reference)
import jax
import jax.numpy as jnp
from jax import lax
from jax.experimental import pallas as pl
from jax.experimental.pallas import tpu as pltpu
from jax.experimental.pallas import tpu_sc as plsc

_N = 10000
_E = 320000
_D = 128
_G = 64
_C = 10

_NC = 2
_NS = 16
_NW = _NC * _NS
_EPW = _E // _NW

_K = 128
_NB = 80
_EPT = _NB * _K
_EPAD = _NW * _EPT

_KD = 80
_NBD = _EPW // _KD

_RS = 640
_RSL = _N - 15 * _RS
_ZC = 80


def _sc_deg_body(dst_hbm, out_hbm, dstv, onesv, zbuf, deg_sh):
    c = lax.axis_index("c")
    s = lax.axis_index("s")
    wid = c * _NS + s
    for k in range(_KD // 16):
        onesv[pl.ds(k * 16, 16)] = jnp.ones((16,), jnp.float32)

    def zb(i, _):
        zbuf[pl.ds(i * 16, 16)] = jnp.zeros((16,), jnp.float32)
        return 0

    lax.fori_loop(0, 125, zb, 0)

    @pl.when(s == 0)
    def _():
        for q in range(5):
            pltpu.sync_copy(zbuf, deg_sh.at[pl.ds(q * 2000, 2000)])

    plsc.subcore_barrier()
    pltpu.sync_copy(dst_hbm.at[wid], dstv)

    def body(j, _):
        pltpu.sync_copy(onesv, deg_sh.at[dstv.at[j]], add=True)
        return 0

    lax.fori_loop(0, _NBD, body, 0)
    plsc.subcore_barrier()

    @pl.when(s == 0)
    def _():
        pltpu.sync_copy(deg_sh, out_hbm.at[c])


def _sc_agg_body(hws_hbm, sd_hbm, out_hbm, sd_a, sd_b, rows_a, rows_b,
                 acc_sh, semi_a, semi_b, semg_a, semg_b):
    c = lax.axis_index("c")
    s = lax.axis_index("s")
    wid = c * _NS + s

    def zb(i, _):
        for k in range(8):
            rows_a[i, pl.ds(k * 16, 16)] = jnp.zeros((16,), jnp.float32)
        return 0

    lax.fori_loop(0, _ZC, zb, 0)
    r0 = s * _RS
    nchunks = jnp.where(s == _NS - 1, _RSL // _ZC, _RS // _ZC)

    def zcopy(q, _):
        pltpu.sync_copy(rows_a.at[pl.ds(0, _ZC)],
                        acc_sh.at[pl.ds(r0 + q * _ZC, _ZC)])
        return 0

    lax.fori_loop(0, nchunks, zcopy, 0)
    plsc.subcore_barrier()

    pltpu.async_copy(sd_hbm.at[wid, 0], sd_a, semi_a)
    pltpu.make_async_copy(sd_hbm.at[wid, 0], sd_a, semi_a).wait()
    pltpu.async_copy(hws_hbm.at[sd_a.at[0]], rows_a, semg_a)
    pltpu.async_copy(sd_hbm.at[wid, 1], sd_b, semi_b)

    def body(j2, _):
        b0 = 2 * j2
        pltpu.make_async_copy(sd_hbm.at[wid, 0], sd_b, semi_b).wait()
        pltpu.async_copy(hws_hbm.at[sd_b.at[0]], rows_b, semg_b)
        pltpu.make_async_copy(hws_hbm.at[pl.ds(0, _K)], rows_a, semg_a).wait()
        pltpu.sync_copy(rows_a, acc_sh.at[sd_a.at[1]], add=True)
        pltpu.async_copy(sd_hbm.at[wid, (b0 + 2) % _NB], sd_a, semi_a)
        pltpu.make_async_copy(hws_hbm.at[pl.ds(0, _K)], rows_b, semg_b).wait()
        pltpu.sync_copy(rows_b, acc_sh.at[sd_b.at[1]], add=True)
        pltpu.async_copy(sd_hbm.at[wid, (b0 + 3) % _NB], sd_b, semi_b)
        pltpu.make_async_copy(sd_hbm.at[wid, 0], sd_a, semi_a).wait()
        pltpu.async_copy(hws_hbm.at[sd_a.at[0]], rows_a, semg_a)
        return 0

    lax.fori_loop(0, _NB // 2, body, 0)
    pltpu.make_async_copy(hws_hbm.at[pl.ds(0, _K)], rows_a, semg_a).wait()
    pltpu.make_async_copy(sd_hbm.at[wid, 0], sd_b, semi_b).wait()
    plsc.subcore_barrier()

    @pl.when(s < _NS - 1)
    def _():
        pltpu.sync_copy(acc_sh.at[pl.ds(r0, _RS)],
                        out_hbm.at[c, pl.ds(r0, _RS)])

    @pl.when(s == _NS - 1)
    def _():
        pltpu.sync_copy(acc_sh.at[pl.ds(r0, _RSL)],
                        out_hbm.at[c, pl.ds(r0, _RSL)])


import functools


@functools.cache
def _get_sc_deg():
    return pl.kernel(
        _sc_deg_body,
        out_type=jax.ShapeDtypeStruct((_NC, _N), jnp.float32),
        mesh=plsc.VectorSubcoreMesh(core_axis_name="c", subcore_axis_name="s",
                                    num_cores=_NC, num_subcores=_NS),
        scratch_types=[
            pltpu.VMEM((_NBD, _KD), jnp.int32),
            pltpu.VMEM((_KD,), jnp.float32),
            pltpu.VMEM((2000,), jnp.float32),
            pltpu.VMEM_SHARED((_N,), jnp.float32),
        ],
    )


@functools.cache
def _get_sc_agg():
    return pl.kernel(
        _sc_agg_body,
        out_type=jax.ShapeDtypeStruct((_NC, _N, _D), jnp.float32),
        mesh=plsc.VectorSubcoreMesh(core_axis_name="c", subcore_axis_name="s",
                                    num_cores=_NC, num_subcores=_NS),
        scratch_types=[
            pltpu.VMEM((2, _K), jnp.int32),
            pltpu.VMEM((2, _K), jnp.int32),
            pltpu.VMEM((_K, _D), jnp.float32),
            pltpu.VMEM((_K, _D), jnp.float32),
            pltpu.VMEM_SHARED((_N + 8, _D), jnp.float32),
            pltpu.SemaphoreType.DMA,
            pltpu.SemaphoreType.DMA,
            pltpu.SemaphoreType.DMA,
            pltpu.SemaphoreType.DMA,
        ],
    )


def _tc_pre_body(x_ref, wemb_ref, bemb_ref, degp_ref, wg0_ref,
                 h0_ref, hws_ref, dinv_ref):
    deg = degp_ref[:, 0:1] + degp_ref[:, 1:2] + 1.0
    dinv = 1.0 / jnp.sqrt(deg)
    h0 = jnp.dot(x_ref[...], wemb_ref[...],
                 preferred_element_type=jnp.float32) + bemb_ref[...]
    h0_ref[...] = h0
    hws_ref[...] = jnp.dot(h0, wg0_ref[...],
                           preferred_element_type=jnp.float32) * dinv
    dinv_ref[...] = dinv


_tc_pre = pl.pallas_call(
    _tc_pre_body,
    out_shape=(
        jax.ShapeDtypeStruct((_N, _D), jnp.float32),
        jax.ShapeDtypeStruct((_N, _D), jnp.float32),
        jax.ShapeDtypeStruct((_N, 1), jnp.float32),
    ),
)


def _bn_relu_res(p0, p1, hws, h_in, dinv, bg, gm, bt):
    agg = (p0 + p1 + hws) * dinv + bg
    mean = jnp.mean(agg, axis=0, keepdims=True)
    var = jnp.mean((agg - mean) ** 2, axis=0, keepdims=True)
    bn = (agg - mean) / jnp.sqrt(var + 1e-5) * gm + bt
    return jnp.maximum(bn, 0.0) + h_in


def _tc_layer_body(p0_ref, p1_ref, hws_ref, h_ref, dinv_ref, bg_ref, gm_ref,
                   bt_ref, wn_ref, h_out_ref, hws_out_ref):
    dinv = dinv_ref[...]
    h = _bn_relu_res(p0_ref[...], p1_ref[...], hws_ref[...], h_ref[...],
                     dinv, bg_ref[...], gm_ref[...], bt_ref[...])
    h_out_ref[...] = h
    hws_out_ref[...] = jnp.dot(h, wn_ref[...],
                               preferred_element_type=jnp.float32) * dinv


_tc_layer = pl.pallas_call(
    _tc_layer_body,
    out_shape=(
        jax.ShapeDtypeStruct((_N, _D), jnp.float32),
        jax.ShapeDtypeStruct((_N, _D), jnp.float32),
    ),
)


def _tc_final_body(p0_ref, p1_ref, hws_ref, h_ref, dinv_ref, bg_ref, gm_ref,
                   bt_ref, batch_ref, wm1_ref, bm1_ref, wm2_ref, bm2_ref,
                   wm3_ref, bm3_ref, out_ref):
    h = _bn_relu_res(p0_ref[...], p1_ref[...], hws_ref[...], h_ref[...],
                     dinv_ref[...], bg_ref[...], gm_ref[...], bt_ref[...])
    gids = lax.broadcasted_iota(jnp.int32, (_G, _N), 0)
    mask = (gids == batch_ref[...]).astype(jnp.float32)
    counts = jnp.sum(mask, axis=1, keepdims=True)
    sums = jnp.dot(mask, h, preferred_element_type=jnp.float32)
    pooled = sums / jnp.maximum(counts, 1.0)
    h1 = jnp.maximum(jnp.dot(pooled, wm1_ref[...],
                             preferred_element_type=jnp.float32)
                     + bm1_ref[...], 0.0)
    h2 = jnp.maximum(jnp.dot(h1, wm2_ref[...],
                             preferred_element_type=jnp.float32)
                     + bm2_ref[...], 0.0)
    out_ref[...] = jnp.dot(h2, wm3_ref[...],
                           preferred_element_type=jnp.float32) + bm3_ref[...]


_tc_final = pl.pallas_call(
    _tc_final_body,
    out_shape=jax.ShapeDtypeStruct((_G, _C), jnp.float32),
)


def kernel(x, edge_index, batch, W_emb, b_emb, Wg, bg, gamma, beta,
           Wm1, bm1, Wm2, bm2, Wm3, bm3):
    src = edge_index[0]
    dst = edge_index[1]
    pad = _EPAD - _E
    src_p = jnp.concatenate([src, jnp.zeros((pad,), src.dtype)])
    dst_p = jnp.concatenate([dst, jnp.full((pad,), _N, dst.dtype)])
    sd = jnp.stack([src_p.reshape(_NW, _NB, _K),
                    dst_p.reshape(_NW, _NB, _K)], axis=2)
    dst3d = dst.reshape(_NW, _NBD, _KD)

    degp = _get_sc_deg()(dst3d)
    h, hws, dinv = _tc_pre(x, W_emb, b_emb.reshape(1, _D),
                           degp.T, Wg[0])
    for i in range(4):
        parts = _get_sc_agg()(hws, sd)
        if i < 3:
            h, hws = _tc_layer(parts[0], parts[1], hws, h, dinv,
                               bg[i].reshape(1, _D), gamma[i].reshape(1, _D),
                               beta[i].reshape(1, _D), Wg[i + 1])
        else:
            logits = _tc_final(parts[0], parts[1], hws, h, dinv,
                               bg[i].reshape(1, _D), gamma[i].reshape(1, _D),
                               beta[i].reshape(1, _D),
                               batch.reshape(1, _N), Wm1,
                               bm1.reshape(1, -1), Wm2, bm2.reshape(1, -1),
                               Wm3, bm3.reshape(1, -1))
    return logits


# speedup vs baseline: 1.2067x; 1.2067x over previous
import jax
import jax.numpy as jnp
from jax import lax
from jax.experimental import pallas as pl
from jax.experimental.pallas import tpu as pltpu
from jax.experimental.pallas import tpu_sc as plsc

_N = 10000
_E = 320000
_D = 128
_G = 64
_C = 10

_NC = 2
_NS = 16
_NW = _NC * _NS
_EPW = _E // _NW

_K = 128
_NB = 80
_EPT = _NB * _K
_EPAD = _NW * _EPT

_KD = 80
_NBD = _EPW // _KD

_RS = 640
_RSL = _N - 15 * _RS
_ZC = 80


def _sc_deg_body(dst_hbm, out_hbm, dstv, onesv, zbuf, deg_sh):
    c = lax.axis_index("c")
    s = lax.axis_index("s")
    wid = c * _NS + s
    for k in range(_KD // 16):
        onesv[pl.ds(k * 16, 16)] = jnp.ones((16,), jnp.float32)

    def zb(i, _):
        zbuf[pl.ds(i * 16, 16)] = jnp.zeros((16,), jnp.float32)
        return 0

    lax.fori_loop(0, 125, zb, 0)

    @pl.when(s == 0)
    def _():
        for q in range(5):
            pltpu.sync_copy(zbuf, deg_sh.at[pl.ds(q * 2000, 2000)])

    plsc.subcore_barrier()
    pltpu.sync_copy(dst_hbm.at[wid], dstv)

    def body(j, _):
        pltpu.sync_copy(onesv, deg_sh.at[dstv.at[j]], add=True)
        return 0

    lax.fori_loop(0, _NBD, body, 0)
    plsc.subcore_barrier()

    @pl.when(s == 0)
    def _():
        pltpu.sync_copy(deg_sh, out_hbm.at[c])


def _sc_agg_body(hws_hbm, pk_hbm, out_hbm, pk_all, si_a, di_a, si_b, di_b,
                 rows_a, rows_b, acc_sh, semg_a, semg_b):
    c = lax.axis_index("c")
    s = lax.axis_index("s")
    wid = c * _NS + s

    def zb(i, _):
        for k in range(8):
            rows_a[i, pl.ds(k * 16, 16)] = jnp.zeros((16,), jnp.float32)
        return 0

    lax.fori_loop(0, _ZC, zb, 0)
    r0 = s * _RS
    nchunks = jnp.where(s == _NS - 1, _RSL // _ZC, _RS // _ZC)

    def zcopy(q, _):
        pltpu.sync_copy(rows_a.at[pl.ds(0, _ZC)],
                        acc_sh.at[pl.ds(r0 + q * _ZC, _ZC)])
        return 0

    lax.fori_loop(0, nchunks, zcopy, 0)
    plsc.subcore_barrier()

    pltpu.sync_copy(pk_hbm.at[wid], pk_all)

    def unpack(jblk, si, di):
        for r in range(8):
            v = pk_all[jblk, pl.ds(16 * r, 16)]
            si[pl.ds(16 * r, 16)] = lax.shift_right_logical(v, 16)
            di[pl.ds(16 * r, 16)] = jnp.bitwise_and(v, 0xFFFF)

    unpack(0, si_a, di_a)
    pltpu.async_copy(hws_hbm.at[si_a], rows_a, semg_a)
    unpack(1, si_b, di_b)

    def body(j2, _):
        b0 = 2 * j2
        pltpu.async_copy(hws_hbm.at[si_b], rows_b, semg_b)
        pltpu.make_async_copy(hws_hbm.at[pl.ds(0, _K)], rows_a, semg_a).wait()
        pltpu.sync_copy(rows_a, acc_sh.at[di_a], add=True)
        unpack((b0 + 2) % _NB, si_a, di_a)
        pltpu.async_copy(hws_hbm.at[si_a], rows_a, semg_a)
        pltpu.make_async_copy(hws_hbm.at[pl.ds(0, _K)], rows_b, semg_b).wait()
        pltpu.sync_copy(rows_b, acc_sh.at[di_b], add=True)
        unpack((b0 + 3) % _NB, si_b, di_b)
        return 0

    lax.fori_loop(0, _NB // 2, body, 0)
    pltpu.make_async_copy(hws_hbm.at[pl.ds(0, _K)], rows_a, semg_a).wait()
    plsc.subcore_barrier()

    @pl.when(s < _NS - 1)
    def _():
        pltpu.sync_copy(acc_sh.at[pl.ds(r0, _RS)],
                        out_hbm.at[c, pl.ds(r0, _RS)])

    @pl.when(s == _NS - 1)
    def _():
        pltpu.sync_copy(acc_sh.at[pl.ds(r0, _RSL)],
                        out_hbm.at[c, pl.ds(r0, _RSL)])


import functools


@functools.cache
def _get_sc_deg():
    return pl.kernel(
        _sc_deg_body,
        out_type=jax.ShapeDtypeStruct((_NC, _N), jnp.float32),
        mesh=plsc.VectorSubcoreMesh(core_axis_name="c", subcore_axis_name="s",
                                    num_cores=_NC, num_subcores=_NS),
        scratch_types=[
            pltpu.VMEM((_NBD, _KD), jnp.int32),
            pltpu.VMEM((_KD,), jnp.float32),
            pltpu.VMEM((2000,), jnp.float32),
            pltpu.VMEM_SHARED((_N,), jnp.float32),
        ],
    )


@functools.cache
def _get_sc_agg():
    return pl.kernel(
        _sc_agg_body,
        out_type=jax.ShapeDtypeStruct((_NC, _N, _D), jnp.float32),
        mesh=plsc.VectorSubcoreMesh(core_axis_name="c", subcore_axis_name="s",
                                    num_cores=_NC, num_subcores=_NS),
        scratch_types=[
            pltpu.VMEM((_NB, _K), jnp.int32),
            pltpu.VMEM((_K,), jnp.int32),
            pltpu.VMEM((_K,), jnp.int32),
            pltpu.VMEM((_K,), jnp.int32),
            pltpu.VMEM((_K,), jnp.int32),
            pltpu.VMEM((_K, _D), jnp.float32),
            pltpu.VMEM((_K, _D), jnp.float32),
            pltpu.VMEM_SHARED((_N + 8, _D), jnp.float32),
            pltpu.SemaphoreType.DMA,
            pltpu.SemaphoreType.DMA,
        ],
    )


def _tc_pre_body(x_ref, wemb_ref, bemb_ref, degp_ref, wg0_ref,
                 h0_ref, hws_ref, dinv_ref):
    deg = degp_ref[:, 0:1] + degp_ref[:, 1:2] + 1.0
    dinv = 1.0 / jnp.sqrt(deg)
    h0 = jnp.dot(x_ref[...], wemb_ref[...],
                 preferred_element_type=jnp.float32) + bemb_ref[...]
    h0_ref[...] = h0
    hws_ref[...] = jnp.dot(h0, wg0_ref[...],
                           preferred_element_type=jnp.float32) * dinv
    dinv_ref[...] = dinv


_tc_pre = pl.pallas_call(
    _tc_pre_body,
    out_shape=(
        jax.ShapeDtypeStruct((_N, _D), jnp.float32),
        jax.ShapeDtypeStruct((_N, _D), jnp.float32),
        jax.ShapeDtypeStruct((_N, 1), jnp.float32),
    ),
)


def _bn_relu_res(p0, p1, hws, h_in, dinv, bg, gm, bt):
    agg = (p0 + p1 + hws) * dinv + bg
    mean = jnp.mean(agg, axis=0, keepdims=True)
    var = jnp.mean((agg - mean) ** 2, axis=0, keepdims=True)
    bn = (agg - mean) / jnp.sqrt(var + 1e-5) * gm + bt
    return jnp.maximum(bn, 0.0) + h_in


def _tc_layer_body(p0_ref, p1_ref, hws_ref, h_ref, dinv_ref, bg_ref, gm_ref,
                   bt_ref, wn_ref, h_out_ref, hws_out_ref):
    dinv = dinv_ref[...]
    h = _bn_relu_res(p0_ref[...], p1_ref[...], hws_ref[...], h_ref[...],
                     dinv, bg_ref[...], gm_ref[...], bt_ref[...])
    h_out_ref[...] = h
    hws_out_ref[...] = jnp.dot(h, wn_ref[...],
                               preferred_element_type=jnp.float32) * dinv


_tc_layer = pl.pallas_call(
    _tc_layer_body,
    out_shape=(
        jax.ShapeDtypeStruct((_N, _D), jnp.float32),
        jax.ShapeDtypeStruct((_N, _D), jnp.float32),
    ),
)


def _tc_final_body(p0_ref, p1_ref, hws_ref, h_ref, dinv_ref, bg_ref, gm_ref,
                   bt_ref, batch_ref, wm1_ref, bm1_ref, wm2_ref, bm2_ref,
                   wm3_ref, bm3_ref, out_ref):
    h = _bn_relu_res(p0_ref[...], p1_ref[...], hws_ref[...], h_ref[...],
                     dinv_ref[...], bg_ref[...], gm_ref[...], bt_ref[...])
    gids = lax.broadcasted_iota(jnp.int32, (_G, _N), 0)
    mask = (gids == batch_ref[...]).astype(jnp.float32)
    counts = jnp.sum(mask, axis=1, keepdims=True)
    sums = jnp.dot(mask, h, preferred_element_type=jnp.float32)
    pooled = sums / jnp.maximum(counts, 1.0)
    h1 = jnp.maximum(jnp.dot(pooled, wm1_ref[...],
                             preferred_element_type=jnp.float32)
                     + bm1_ref[...], 0.0)
    h2 = jnp.maximum(jnp.dot(h1, wm2_ref[...],
                             preferred_element_type=jnp.float32)
                     + bm2_ref[...], 0.0)
    out_ref[...] = jnp.dot(h2, wm3_ref[...],
                           preferred_element_type=jnp.float32) + bm3_ref[...]


_tc_final = pl.pallas_call(
    _tc_final_body,
    out_shape=jax.ShapeDtypeStruct((_G, _C), jnp.float32),
)


def kernel(x, edge_index, batch, W_emb, b_emb, Wg, bg, gamma, beta,
           Wm1, bm1, Wm2, bm2, Wm3, bm3):
    src = edge_index[0]
    dst = edge_index[1]
    pad = _EPAD - _E
    src_p = jnp.concatenate([src, jnp.zeros((pad,), src.dtype)])
    dst_p = jnp.concatenate([dst, jnp.full((pad,), _N, dst.dtype)])
    packed = jnp.bitwise_or(jnp.left_shift(src_p, 16), dst_p)
    pk3 = packed.reshape(_NW, _NB, _K)
    dst3d = dst.reshape(_NW, _NBD, _KD)

    degp = _get_sc_deg()(dst3d)
    h, hws, dinv = _tc_pre(x, W_emb, b_emb.reshape(1, _D),
                           degp.T, Wg[0])
    for i in range(4):
        parts = _get_sc_agg()(hws, pk3)
        if i < 3:
            h, hws = _tc_layer(parts[0], parts[1], hws, h, dinv,
                               bg[i].reshape(1, _D), gamma[i].reshape(1, _D),
                               beta[i].reshape(1, _D), Wg[i + 1])
        else:
            logits = _tc_final(parts[0], parts[1], hws, h, dinv,
                               bg[i].reshape(1, _D), gamma[i].reshape(1, _D),
                               beta[i].reshape(1, _D),
                               batch.reshape(1, _N), Wm1,
                               bm1.reshape(1, -1), Wm2, bm2.reshape(1, -1),
                               Wm3, bm3.reshape(1, -1))
    return logits


# speedup vs baseline: 2.4656x; 2.0433x over previous
import jax
import jax.numpy as jnp
from jax import lax
from jax.experimental import pallas as pl
from jax.experimental.pallas import tpu as pltpu
from jax.experimental.pallas import tpu_sc as plsc

_N = 10000
_E = 320000
_D = 128
_G = 64
_C = 10

_NC = 2
_NS = 16
_NW = _NC * _NS
_EPW = _E // _NW

_K = 100
_NB = _EPW // _K

_KD = 80
_NBD = _EPW // _KD

_RS = 640
_RSL = _N - 15 * _RS
_ZC = 80


def _sc_deg_body(dst_hbm, out_hbm, dstv, onesv, zbuf, deg_sh):
    c = lax.axis_index("c")
    s = lax.axis_index("s")
    wid = c * _NS + s
    for k in range(_KD // 16):
        onesv[pl.ds(k * 16, 16)] = jnp.ones((16,), jnp.float32)

    def zb(i, _):
        zbuf[pl.ds(i * 16, 16)] = jnp.zeros((16,), jnp.float32)
        return 0

    lax.fori_loop(0, 125, zb, 0)

    @pl.when(s == 0)
    def _():
        for q in range(5):
            pltpu.sync_copy(zbuf, deg_sh.at[pl.ds(q * 2000, 2000)])

    plsc.subcore_barrier()
    pltpu.sync_copy(dst_hbm.at[wid], dstv)

    def body(j, _):
        pltpu.sync_copy(onesv, deg_sh.at[dstv.at[j]], add=True)
        return 0

    lax.fori_loop(0, _NBD, body, 0)
    plsc.subcore_barrier()

    @pl.when(s == 0)
    def _():
        pltpu.sync_copy(deg_sh, out_hbm.at[c])


def _sc_agg_body(hws_hbm, src_hbm, dst_hbm, out_hbm, srcv, dstv, rows,
                 acc_sh, sem):
    c = lax.axis_index("c")
    s = lax.axis_index("s")
    wid = c * _NS + s

    def zb(i, _):
        for k in range(8):
            rows[i, pl.ds(k * 16, 16)] = jnp.zeros((16,), jnp.float32)
        return 0

    lax.fori_loop(0, _ZC, zb, 0)
    r0 = s * _RS
    nchunks = jnp.where(s == _NS - 1, _RSL // _ZC, _RS // _ZC)

    def zcopy(q, _):
        pltpu.sync_copy(rows.at[pl.ds(0, _ZC)],
                        acc_sh.at[pl.ds(r0 + q * _ZC, _ZC)])
        return 0

    lax.fori_loop(0, nchunks, zcopy, 0)
    plsc.subcore_barrier()

    pltpu.sync_copy(src_hbm.at[wid], srcv)
    pltpu.sync_copy(dst_hbm.at[wid], dstv)

    def body(j, _):
        pltpu.async_copy(hws_hbm.at[srcv.at[j]], rows, sem).wait()
        pltpu.sync_copy(rows, acc_sh.at[dstv.at[j]], add=True)
        return 0

    lax.fori_loop(0, _NB, body, 0)
    plsc.subcore_barrier()

    @pl.when(s < _NS - 1)
    def _():
        pltpu.sync_copy(acc_sh.at[pl.ds(r0, _RS)],
                        out_hbm.at[c, pl.ds(r0, _RS)])

    @pl.when(s == _NS - 1)
    def _():
        pltpu.sync_copy(acc_sh.at[pl.ds(r0, _RSL)],
                        out_hbm.at[c, pl.ds(r0, _RSL)])


import functools


@functools.cache
def _get_sc_deg():
    return pl.kernel(
        _sc_deg_body,
        out_type=jax.ShapeDtypeStruct((_NC, _N), jnp.float32),
        mesh=plsc.VectorSubcoreMesh(core_axis_name="c", subcore_axis_name="s",
                                    num_cores=_NC, num_subcores=_NS),
        scratch_types=[
            pltpu.VMEM((_NBD, _KD), jnp.int32),
            pltpu.VMEM((_KD,), jnp.float32),
            pltpu.VMEM((2000,), jnp.float32),
            pltpu.VMEM_SHARED((_N,), jnp.float32),
        ],
    )


@functools.cache
def _get_sc_agg():
    return pl.kernel(
        _sc_agg_body,
        out_type=jax.ShapeDtypeStruct((_NC, _N, _D), jnp.float32),
        mesh=plsc.VectorSubcoreMesh(core_axis_name="c", subcore_axis_name="s",
                                    num_cores=_NC, num_subcores=_NS),
        scratch_types=[
            pltpu.VMEM((_NB, _K), jnp.int32),
            pltpu.VMEM((_NB, _K), jnp.int32),
            pltpu.VMEM((_K, _D), jnp.float32),
            pltpu.VMEM_SHARED((_N, _D), jnp.float32),
            pltpu.SemaphoreType.DMA,
        ],
    )


def _tc_pre_body(x_ref, wemb_ref, bemb_ref, degp_ref, wg0_ref,
                 h0_ref, hws_ref, dinv_ref):
    deg = degp_ref[:, 0:1] + degp_ref[:, 1:2] + 1.0
    dinv = 1.0 / jnp.sqrt(deg)
    h0 = jnp.dot(x_ref[...], wemb_ref[...],
                 preferred_element_type=jnp.float32) + bemb_ref[...]
    h0_ref[...] = h0
    hws_ref[...] = jnp.dot(h0, wg0_ref[...],
                           preferred_element_type=jnp.float32) * dinv
    dinv_ref[...] = dinv


_tc_pre = pl.pallas_call(
    _tc_pre_body,
    out_shape=(
        jax.ShapeDtypeStruct((_N, _D), jnp.float32),
        jax.ShapeDtypeStruct((_N, _D), jnp.float32),
        jax.ShapeDtypeStruct((_N, 1), jnp.float32),
    ),
)


def _bn_relu_res(p0, p1, hws, h_in, dinv, bg, gm, bt):
    agg = (p0 + p1 + hws) * dinv + bg
    mean = jnp.mean(agg, axis=0, keepdims=True)
    var = jnp.mean((agg - mean) ** 2, axis=0, keepdims=True)
    bn = (agg - mean) / jnp.sqrt(var + 1e-5) * gm + bt
    return jnp.maximum(bn, 0.0) + h_in


def _tc_layer_body(p0_ref, p1_ref, hws_ref, h_ref, dinv_ref, bg_ref, gm_ref,
                   bt_ref, wn_ref, h_out_ref, hws_out_ref):
    dinv = dinv_ref[...]
    h = _bn_relu_res(p0_ref[...], p1_ref[...], hws_ref[...], h_ref[...],
                     dinv, bg_ref[...], gm_ref[...], bt_ref[...])
    h_out_ref[...] = h
    hws_out_ref[...] = jnp.dot(h, wn_ref[...],
                               preferred_element_type=jnp.float32) * dinv


_tc_layer = pl.pallas_call(
    _tc_layer_body,
    out_shape=(
        jax.ShapeDtypeStruct((_N, _D), jnp.float32),
        jax.ShapeDtypeStruct((_N, _D), jnp.float32),
    ),
)


def _tc_final_body(p0_ref, p1_ref, hws_ref, h_ref, dinv_ref, bg_ref, gm_ref,
                   bt_ref, batch_ref, wm1_ref, bm1_ref, wm2_ref, bm2_ref,
                   wm3_ref, bm3_ref, out_ref):
    h = _bn_relu_res(p0_ref[...], p1_ref[...], hws_ref[...], h_ref[...],
                     dinv_ref[...], bg_ref[...], gm_ref[...], bt_ref[...])
    gids = lax.broadcasted_iota(jnp.int32, (_G, _N), 0)
    mask = (gids == batch_ref[...]).astype(jnp.float32)
    counts = jnp.sum(mask, axis=1, keepdims=True)
    sums = jnp.dot(mask, h, preferred_element_type=jnp.float32)
    pooled = sums / jnp.maximum(counts, 1.0)
    h1 = jnp.maximum(jnp.dot(pooled, wm1_ref[...],
                             preferred_element_type=jnp.float32)
                     + bm1_ref[...], 0.0)
    h2 = jnp.maximum(jnp.dot(h1, wm2_ref[...],
                             preferred_element_type=jnp.float32)
                     + bm2_ref[...], 0.0)
    out_ref[...] = jnp.dot(h2, wm3_ref[...],
                           preferred_element_type=jnp.float32) + bm3_ref[...]


_tc_final = pl.pallas_call(
    _tc_final_body,
    out_shape=jax.ShapeDtypeStruct((_G, _C), jnp.float32),
)


def kernel(x, edge_index, batch, W_emb, b_emb, Wg, bg, gamma, beta,
           Wm1, bm1, Wm2, bm2, Wm3, bm3):
    src = edge_index[0]
    dst = edge_index[1]
    src3 = src.reshape(_NW, _NB, _K)
    dst3 = dst.reshape(_NW, _NB, _K)
    dst3d = dst.reshape(_NW, _NBD, _KD)

    degp = _get_sc_deg()(dst3d)
    h, hws, dinv = _tc_pre(x, W_emb, b_emb.reshape(1, _D),
                           degp.T, Wg[0])
    for i in range(4):
        parts = _get_sc_agg()(hws, src3, dst3)
        if i < 3:
            h, hws = _tc_layer(parts[0], parts[1], hws, h, dinv,
                               bg[i].reshape(1, _D), gamma[i].reshape(1, _D),
                               beta[i].reshape(1, _D), Wg[i + 1])
        else:
            logits = _tc_final(parts[0], parts[1], hws, h, dinv,
                               bg[i].reshape(1, _D), gamma[i].reshape(1, _D),
                               beta[i].reshape(1, _D),
                               batch.reshape(1, _N), Wm1,
                               bm1.reshape(1, -1), Wm2, bm2.reshape(1, -1),
                               Wm3, bm3.reshape(1, -1))
    return logits


# speedup vs baseline: 2.6366x; 1.0694x over previous
import jax
import jax.numpy as jnp
from jax import lax
from jax.experimental import pallas as pl
from jax.experimental.pallas import tpu as pltpu
from jax.experimental.pallas import tpu_sc as plsc

_N = 10000
_E = 320000
_D = 128
_G = 64
_C = 10

_NC = 2
_NS = 16
_NW = _NC * _NS
_EPW = _E // _NW

_K = 125
_NB = _EPW // _K

_KD = 80
_NBD = _EPW // _KD

_RS = 640
_RSL = _N - 15 * _RS
_ZC = 80


def _sc_deg_body(dst_hbm, out_hbm, dstv, onesv, zbuf, deg_sh):
    c = lax.axis_index("c")
    s = lax.axis_index("s")
    wid = c * _NS + s
    for k in range(_KD // 16):
        onesv[pl.ds(k * 16, 16)] = jnp.ones((16,), jnp.float32)

    def zb(i, _):
        zbuf[pl.ds(i * 16, 16)] = jnp.zeros((16,), jnp.float32)
        return 0

    lax.fori_loop(0, 125, zb, 0)

    @pl.when(s == 0)
    def _():
        for q in range(5):
            pltpu.sync_copy(zbuf, deg_sh.at[pl.ds(q * 2000, 2000)])

    plsc.subcore_barrier()
    pltpu.sync_copy(dst_hbm.at[wid], dstv)

    def body(j, _):
        pltpu.sync_copy(onesv, deg_sh.at[dstv.at[j]], add=True)
        return 0

    lax.fori_loop(0, _NBD, body, 0)
    plsc.subcore_barrier()

    @pl.when(s == 0)
    def _():
        pltpu.sync_copy(deg_sh, out_hbm.at[c])


def _sc_agg_body(hws_hbm, src_hbm, dst_hbm, out_hbm, srcv, dstv, rows,
                 acc_sh, sem):
    c = lax.axis_index("c")
    s = lax.axis_index("s")
    wid = c * _NS + s

    def zb(i, _):
        for k in range(8):
            rows[i, pl.ds(k * 16, 16)] = jnp.zeros((16,), jnp.float32)
        return 0

    lax.fori_loop(0, _ZC, zb, 0)
    r0 = s * _RS
    nchunks = jnp.where(s == _NS - 1, _RSL // _ZC, _RS // _ZC)

    def zcopy(q, _):
        pltpu.sync_copy(rows.at[pl.ds(0, _ZC)],
                        acc_sh.at[pl.ds(r0 + q * _ZC, _ZC)])
        return 0

    lax.fori_loop(0, nchunks, zcopy, 0)
    plsc.subcore_barrier()

    pltpu.sync_copy(src_hbm.at[wid], srcv)
    pltpu.sync_copy(dst_hbm.at[wid], dstv)

    def body(j, _):
        pltpu.async_copy(hws_hbm.at[srcv.at[j]], rows, sem).wait()
        pltpu.sync_copy(rows, acc_sh.at[dstv.at[j]], add=True)
        return 0

    lax.fori_loop(0, _NB, body, 0)
    plsc.subcore_barrier()

    @pl.when(s < _NS - 1)
    def _():
        pltpu.sync_copy(acc_sh.at[pl.ds(r0, _RS)],
                        out_hbm.at[c, pl.ds(r0, _RS)])

    @pl.when(s == _NS - 1)
    def _():
        pltpu.sync_copy(acc_sh.at[pl.ds(r0, _RSL)],
                        out_hbm.at[c, pl.ds(r0, _RSL)])


import functools


@functools.cache
def _get_sc_deg():
    return pl.kernel(
        _sc_deg_body,
        out_type=jax.ShapeDtypeStruct((_NC, _N), jnp.float32),
        mesh=plsc.VectorSubcoreMesh(core_axis_name="c", subcore_axis_name="s",
                                    num_cores=_NC, num_subcores=_NS),
        scratch_types=[
            pltpu.VMEM((_NBD, _KD), jnp.int32),
            pltpu.VMEM((_KD,), jnp.float32),
            pltpu.VMEM((2000,), jnp.float32),
            pltpu.VMEM_SHARED((_N,), jnp.float32),
        ],
    )


@functools.cache
def _get_sc_agg():
    return pl.kernel(
        _sc_agg_body,
        out_type=jax.ShapeDtypeStruct((_NC, _N, _D), jnp.float32),
        mesh=plsc.VectorSubcoreMesh(core_axis_name="c", subcore_axis_name="s",
                                    num_cores=_NC, num_subcores=_NS),
        scratch_types=[
            pltpu.VMEM((_NB, _K), jnp.int32),
            pltpu.VMEM((_NB, _K), jnp.int32),
            pltpu.VMEM((_K, _D), jnp.float32),
            pltpu.VMEM_SHARED((_N, _D), jnp.float32),
            pltpu.SemaphoreType.DMA,
        ],
    )


def _tc_pre_body(x_ref, wemb_ref, bemb_ref, degp_ref, wg0_ref,
                 h0_ref, hws_ref, dinv_ref):
    deg = degp_ref[:, 0:1] + degp_ref[:, 1:2] + 1.0
    dinv = 1.0 / jnp.sqrt(deg)
    h0 = jnp.dot(x_ref[...], wemb_ref[...],
                 preferred_element_type=jnp.float32) + bemb_ref[...]
    h0_ref[...] = h0
    hws_ref[...] = jnp.dot(h0, wg0_ref[...],
                           preferred_element_type=jnp.float32) * dinv
    dinv_ref[...] = dinv


_tc_pre = pl.pallas_call(
    _tc_pre_body,
    out_shape=(
        jax.ShapeDtypeStruct((_N, _D), jnp.float32),
        jax.ShapeDtypeStruct((_N, _D), jnp.float32),
        jax.ShapeDtypeStruct((_N, 1), jnp.float32),
    ),
)


def _bn_relu_res(p0, p1, hws, h_in, dinv, bg, gm, bt):
    agg = (p0 + p1 + hws) * dinv + bg
    mean = jnp.mean(agg, axis=0, keepdims=True)
    var = jnp.mean((agg - mean) ** 2, axis=0, keepdims=True)
    bn = (agg - mean) / jnp.sqrt(var + 1e-5) * gm + bt
    return jnp.maximum(bn, 0.0) + h_in


def _tc_layer_body(p0_ref, p1_ref, hws_ref, h_ref, dinv_ref, bg_ref, gm_ref,
                   bt_ref, wn_ref, h_out_ref, hws_out_ref):
    dinv = dinv_ref[...]
    h = _bn_relu_res(p0_ref[...], p1_ref[...], hws_ref[...], h_ref[...],
                     dinv, bg_ref[...], gm_ref[...], bt_ref[...])
    h_out_ref[...] = h
    hws_out_ref[...] = jnp.dot(h, wn_ref[...],
                               preferred_element_type=jnp.float32) * dinv


_tc_layer = pl.pallas_call(
    _tc_layer_body,
    out_shape=(
        jax.ShapeDtypeStruct((_N, _D), jnp.float32),
        jax.ShapeDtypeStruct((_N, _D), jnp.float32),
    ),
)


def _tc_final_body(p0_ref, p1_ref, hws_ref, h_ref, dinv_ref, bg_ref, gm_ref,
                   bt_ref, batch_ref, wm1_ref, bm1_ref, wm2_ref, bm2_ref,
                   wm3_ref, bm3_ref, out_ref):
    h = _bn_relu_res(p0_ref[...], p1_ref[...], hws_ref[...], h_ref[...],
                     dinv_ref[...], bg_ref[...], gm_ref[...], bt_ref[...])
    gids = lax.broadcasted_iota(jnp.int32, (_G, _N), 0)
    mask = (gids == batch_ref[...]).astype(jnp.float32)
    counts = jnp.sum(mask, axis=1, keepdims=True)
    sums = jnp.dot(mask, h, preferred_element_type=jnp.float32)
    pooled = sums / jnp.maximum(counts, 1.0)
    h1 = jnp.maximum(jnp.dot(pooled, wm1_ref[...],
                             preferred_element_type=jnp.float32)
                     + bm1_ref[...], 0.0)
    h2 = jnp.maximum(jnp.dot(h1, wm2_ref[...],
                             preferred_element_type=jnp.float32)
                     + bm2_ref[...], 0.0)
    out_ref[...] = jnp.dot(h2, wm3_ref[...],
                           preferred_element_type=jnp.float32) + bm3_ref[...]


_tc_final = pl.pallas_call(
    _tc_final_body,
    out_shape=jax.ShapeDtypeStruct((_G, _C), jnp.float32),
)


def kernel(x, edge_index, batch, W_emb, b_emb, Wg, bg, gamma, beta,
           Wm1, bm1, Wm2, bm2, Wm3, bm3):
    src = edge_index[0]
    dst = edge_index[1]
    src3 = src.reshape(_NW, _NB, _K)
    dst3 = dst.reshape(_NW, _NB, _K)
    dst3d = dst.reshape(_NW, _NBD, _KD)

    degp = _get_sc_deg()(dst3d)
    h, hws, dinv = _tc_pre(x, W_emb, b_emb.reshape(1, _D),
                           degp.T, Wg[0])
    for i in range(4):
        parts = _get_sc_agg()(hws, src3, dst3)
        if i < 3:
            h, hws = _tc_layer(parts[0], parts[1], hws, h, dinv,
                               bg[i].reshape(1, _D), gamma[i].reshape(1, _D),
                               beta[i].reshape(1, _D), Wg[i + 1])
        else:
            logits = _tc_final(parts[0], parts[1], hws, h, dinv,
                               bg[i].reshape(1, _D), gamma[i].reshape(1, _D),
                               beta[i].reshape(1, _D),
                               batch.reshape(1, _N), Wm1,
                               bm1.reshape(1, -1), Wm2, bm2.reshape(1, -1),
                               Wm3, bm3.reshape(1, -1))
    return logits
